# baseline (speedup 1.0000x reference)
import functools

import jax
import jax.numpy as jnp
from jax import lax
from jax.experimental import pallas as pl
from jax.experimental.pallas import tpu as pltpu
from jax.experimental.pallas import tpu_sc as plsc

B = 16384
NFEAT = 1000000
NCONT = 13
NCATE = 26
NTOT = NCONT + NCATE
D = 32
NC = 2
NS = 16
BPT = B // (NC * NS)

_mesh = plsc.VectorSubcoreMesh(core_axis_name="c", subcore_axis_name="s")


@functools.partial(
    pl.kernel,
    mesh=_mesh,
    compiler_params=pltpu.CompilerParams(use_tc_tiling_on_sc=False),
    out_type=jax.ShapeDtypeStruct((NTOT, D, B), jnp.float32),
    scratch_types=[
        pltpu.VMEM((NCATE, BPT), jnp.int32),
        pltpu.VMEM((NCONT, BPT), jnp.float32),
        pltpu.VMEM((16,), jnp.int32),
        pltpu.VMEM((16,), jnp.float32),
        pltpu.VMEM((NTOT, BPT), jnp.float32),
        pltpu.VMEM_SHARED((NFEAT,), jnp.float32),
        pltpu.SemaphoreType.DMA,
        pltpu.SemaphoreType.DMA,
    ],
)
def _emb_lookup(
    cvt_hbm, ucit_hbm, tabt_hbm, cidx_hbm, out_hbm,
    idxs, cvs, cidx_v, cont_sc, stage, row_sp, sem, sem2,
):
    s = lax.axis_index("s")
    c = lax.axis_index("c")
    btile = c * (NS * BPT) + s * BPT

    pltpu.sync_copy(ucit_hbm.at[:, pl.ds(btile, BPT)], idxs)
    pltpu.sync_copy(cvt_hbm.at[:, pl.ds(btile, BPT)], cvs)
    pltpu.sync_copy(cidx_hbm, cidx_v.at[pl.ds(0, NCONT)])

    def per_d(d, carry):
        @pl.when(s == 0)
        def _():
            pltpu.sync_copy(tabt_hbm.at[d], row_sp)

        plsc.subcore_barrier()

        pltpu.async_copy(
            row_sp.at[cidx_v.at[pl.ds(0, NCONT)]],
            cont_sc.at[pl.ds(0, NCONT)],
            sem,
        ).wait()
        cvec = cont_sc[pl.ds(0, 16)]
        scal = [cvec[j] for j in range(NCONT)]

        def mulk(k, c2):
            o = k * 16
            for j in range(NCONT):
                stage[j, pl.ds(o, 16)] = cvs[j, pl.ds(o, 16)] * scal[j]
            return c2

        lax.fori_loop(0, BPT // 16, mulk, 0)

        cps = [
            pltpu.async_copy(row_sp.at[idxs.at[jj]], stage.at[NCONT + jj], sem2)
            for jj in range(NCATE)
        ]
        for cp in cps:
            cp.wait()

        pltpu.sync_copy(stage, out_hbm.at[:, d, pl.ds(btile, BPT)])
        plsc.subcore_barrier()
        return carry

    lax.fori_loop(0, D, per_d, 0)


def kernel(continuous_value, universal_category_index, emb_table, cont_idx):
    out = _emb_lookup(
        continuous_value.T,
        universal_category_index.astype(jnp.int32).T,
        emb_table.T,
        cont_idx.astype(jnp.int32),
    )
    return out.transpose(2, 0, 1)


# speedup vs baseline: 8.2114x; 8.2114x over previous
import functools

import jax
import jax.numpy as jnp
from jax import lax
from jax.experimental import pallas as pl
from jax.experimental.pallas import tpu as pltpu
from jax.experimental.pallas import tpu_sc as plsc

B = 16384
NFEAT = 1000000
NFPAD = 1000064
NCONT = 13
NCATE = 26
NTOT = NCONT + NCATE
D = 32
NC = 2
NS = 16
BPT = B // (NC * NS)

_mesh = plsc.VectorSubcoreMesh(core_axis_name="c", subcore_axis_name="s")


@functools.partial(
    pl.kernel,
    mesh=_mesh,
    compiler_params=pltpu.CompilerParams(use_tc_tiling_on_sc=True),
    out_type=jax.ShapeDtypeStruct((NTOT, D, B), jnp.float32),
    scratch_types=[
        pltpu.VMEM((NCATE, 1, BPT), jnp.int32),
        pltpu.VMEM((NCONT, 1, BPT), jnp.float32),
        pltpu.VMEM((16,), jnp.int32),
        pltpu.VMEM((16,), jnp.float32),
        pltpu.VMEM((NTOT, 1, BPT), jnp.float32),
        pltpu.VMEM_SHARED((1, NFEAT), jnp.float32),
        pltpu.SemaphoreType.DMA,
        pltpu.SemaphoreType.DMA,
    ],
)
def _emb_lookup(
    cvt_hbm, ucit_hbm, tabt_hbm, cidx_hbm, out_hbm,
    idxs, cvs, cidx_v, cont_sc, stage, row_sp, sem, sem2,
):
    s = lax.axis_index("s")
    c = lax.axis_index("c")
    btile = c * (NS * BPT) + s * BPT

    pltpu.sync_copy(ucit_hbm.at[:, pl.ds(btile, BPT)], idxs.at[:, 0, :])
    pltpu.sync_copy(cvt_hbm.at[:, pl.ds(btile, BPT)], cvs.at[:, 0, :])
    pltpu.sync_copy(cidx_hbm, cidx_v.at[pl.ds(0, NCONT)])

    def per_d(d, carry):
        @pl.when(s == 0)
        def _():
            pltpu.sync_copy(tabt_hbm.at[pl.ds(d, 1), :], row_sp)

        plsc.subcore_barrier()

        pltpu.async_copy(
            row_sp.at[0].at[cidx_v.at[pl.ds(0, NCONT)]],
            cont_sc.at[pl.ds(0, NCONT)],
            sem,
        ).wait()
        cvec = cont_sc[pl.ds(0, 16)]
        scal = [cvec[j] for j in range(NCONT)]

        def mulk(k, c2):
            o = k * 16
            for j in range(NCONT):
                stage[j, 0, pl.ds(o, 16)] = cvs[j, 0, pl.ds(o, 16)] * scal[j]
            return c2

        lax.fori_loop(0, BPT // 16, mulk, 0)

        cps = [
            pltpu.async_copy(
                row_sp.at[0].at[idxs.at[jj, 0]], stage.at[NCONT + jj, 0], sem2
            )
            for jj in range(NCATE)
        ]
        for cp in cps:
            cp.wait()

        pltpu.sync_copy(
            stage, out_hbm.at[:, pl.ds(d, 1), pl.ds(btile, BPT)]
        )
        plsc.subcore_barrier()
        return carry

    lax.fori_loop(0, D, per_d, 0)


def kernel(continuous_value, universal_category_index, emb_table, cont_idx):
    out = _emb_lookup(
        continuous_value.T,
        universal_category_index.astype(jnp.int32).T,
        emb_table.T,
        cont_idx.astype(jnp.int32),
    )
    return out.transpose(2, 0, 1)


# speedup vs baseline: 9.0346x; 1.1003x over previous
import functools

import jax
import jax.numpy as jnp
from jax import lax
from jax.experimental import pallas as pl
from jax.experimental.pallas import tpu as pltpu
from jax.experimental.pallas import tpu_sc as plsc

B = 16384
NFEAT = 1000000
NCONT = 13
NCATE = 26
NTOT = NCONT + NCATE
D = 32
NC = 2
NS = 16
BPT = B // (NC * NS)

_mesh = plsc.VectorSubcoreMesh(core_axis_name="c", subcore_axis_name="s")


@functools.partial(
    pl.kernel,
    mesh=_mesh,
    compiler_params=pltpu.CompilerParams(use_tc_tiling_on_sc=True),
    out_type=jax.ShapeDtypeStruct((NTOT, D, B), jnp.float32),
    scratch_types=[
        pltpu.VMEM((NCATE, 1, BPT), jnp.int32),
        pltpu.VMEM((NCONT, 1, BPT), jnp.float32),
        pltpu.VMEM((16,), jnp.int32),
        pltpu.VMEM((16,), jnp.float32),
        pltpu.VMEM((2, NTOT, 1, BPT), jnp.float32),
        pltpu.VMEM_SHARED((1, NFEAT), jnp.float32),
        pltpu.SemaphoreType.DMA,
        pltpu.SemaphoreType.DMA,
        pltpu.SemaphoreType.DMA,
        pltpu.SemaphoreType.DMA,
        pltpu.SemaphoreType.DMA,
    ],
)
def _emb_lookup(
    cvt_hbm, ucit_hbm, tabt_hbm, cidx_hbm, out_hbm,
    idxs, cvs, cidx_v, cont_sc, stage, row2,
    sem_s, sem_c, sem_g, sem_o0, sem_o1,
):
    s = lax.axis_index("s")
    c = lax.axis_index("c")
    btile = c * (NS * BPT) + s * BPT

    pltpu.sync_copy(ucit_hbm.at[:, pl.ds(btile, BPT)], idxs.at[:, 0, :])
    pltpu.sync_copy(cvt_hbm.at[:, pl.ds(btile, BPT)], cvs.at[:, 0, :])
    pltpu.sync_copy(cidx_hbm, cidx_v.at[pl.ds(0, NCONT)])
    lane = lax.iota(jnp.int32, 16)
    cidx_v[pl.ds(0, 16)] = jnp.where(lane < NCONT, cidx_v[pl.ds(0, 16)], 0)

    row = row2.at[0]

    @pl.when(s == 0)
    def _():
        pltpu.async_copy(tabt_hbm.at[pl.ds(0, 1), :], row2, sem_s)

    def process(d, sbuf, sem_o):
        @pl.when(d >= 2)
        def _():
            pltpu.make_async_copy(
                sbuf,
                out_hbm.at[:, pl.ds(0, 1), pl.ds(btile, BPT)],
                sem_o,
            ).wait()

        cpc = pltpu.async_copy(row.at[cidx_v], cont_sc, sem_c)

        def fire(jj, c2):
            pltpu.async_copy(
                row.at[idxs.at[jj, 0]], sbuf.at[NCONT + jj, 0], sem_g
            )
            return c2

        lax.fori_loop(0, NCATE, fire, 0)

        cpc.wait()
        cvec = cont_sc[pl.ds(0, 16)]
        scal = [cvec[j] for j in range(NCONT)]

        def mulk(k, c2):
            o = k * 16
            for j in range(NCONT):
                sbuf[j, 0, pl.ds(o, 16)] = cvs[j, 0, pl.ds(o, 16)] * scal[j]
            return c2

        lax.fori_loop(0, BPT // 16, mulk, 0)

        def drain(jj, c2):
            pltpu.make_async_copy(
                row.at[idxs.at[0, 0]], sbuf.at[NCONT, 0], sem_g
            ).wait()
            return c2

        lax.fori_loop(0, NCATE, drain, 0)

        plsc.subcore_barrier()

        @pl.when(jnp.logical_and(s == 0, d + 1 < D))
        def _():
            nd = jnp.minimum(d + 1, D - 1)
            pltpu.async_copy(tabt_hbm.at[pl.ds(nd, 1), :], row2, sem_s)

        pltpu.async_copy(
            sbuf, out_hbm.at[:, pl.ds(d, 1), pl.ds(btile, BPT)], sem_o
        )

    def per_pair(p, carry):
        d0 = 2 * p
        d1 = 2 * p + 1

        @pl.when(s == 0)
        def _():
            pltpu.make_async_copy(
                tabt_hbm.at[pl.ds(d0, 1), :], row2, sem_s
            ).wait()

        plsc.subcore_barrier()
        process(d0, stage.at[0], sem_o0)

        @pl.when(s == 0)
        def _():
            pltpu.make_async_copy(
                tabt_hbm.at[pl.ds(d1, 1), :], row2, sem_s
            ).wait()

        plsc.subcore_barrier()
        process(d1, stage.at[1], sem_o1)
        return carry

    lax.fori_loop(0, D // 2, per_pair, 0)

    pltpu.make_async_copy(
        stage.at[0], out_hbm.at[:, pl.ds(0, 1), pl.ds(btile, BPT)], sem_o0
    ).wait()
    pltpu.make_async_copy(
        stage.at[1], out_hbm.at[:, pl.ds(0, 1), pl.ds(btile, BPT)], sem_o1
    ).wait()


def kernel(continuous_value, universal_category_index, emb_table, cont_idx):
    out = _emb_lookup(
        continuous_value.T,
        universal_category_index.astype(jnp.int32).T,
        emb_table.T,
        cont_idx.astype(jnp.int32),
    )
    return out.transpose(2, 0, 1)


# speedup vs baseline: 9.0602x; 1.0028x over previous
import functools

import jax
import jax.numpy as jnp
from jax import lax
from jax.experimental import pallas as pl
from jax.experimental.pallas import tpu as pltpu
from jax.experimental.pallas import tpu_sc as plsc

B = 16384
NFEAT = 1000000
NCONT = 13
NCATE = 26
NTOT = NCONT + NCATE
D = 32
NC = 2
NS = 16
BPT = B // (NC * NS)
NFPAD = 1000064
NSTREAM = 12
CHK = 83328

_mesh = plsc.VectorSubcoreMesh(core_axis_name="c", subcore_axis_name="s")


@functools.partial(
    pl.kernel,
    mesh=_mesh,
    compiler_params=pltpu.CompilerParams(use_tc_tiling_on_sc=True),
    out_type=jax.ShapeDtypeStruct((NTOT, D, B), jnp.float32),
    scratch_types=[
        pltpu.VMEM((NCATE, 1, BPT), jnp.int32),
        pltpu.VMEM((NCONT, 1, BPT), jnp.float32),
        pltpu.VMEM((16,), jnp.int32),
        pltpu.VMEM((16,), jnp.float32),
        pltpu.VMEM((2, NTOT, 1, BPT), jnp.float32),
        pltpu.VMEM_SHARED((1, NFPAD), jnp.float32),
        pltpu.SemaphoreType.DMA,
        pltpu.SemaphoreType.DMA,
        pltpu.SemaphoreType.DMA,
        pltpu.SemaphoreType.DMA,
        pltpu.SemaphoreType.DMA,
    ],
)
def _emb_lookup(
    cvt_hbm, ucit_hbm, tabt_hbm, tail_hbm, cidx_hbm, out_hbm,
    idxs, cvs, cidx_v, cont_sc, stage, row2,
    sem_s, sem_c, sem_g, sem_o0, sem_o1,
):
    s = lax.axis_index("s")
    c = lax.axis_index("c")
    btile = c * (NS * BPT) + s * BPT

    pltpu.sync_copy(ucit_hbm.at[:, pl.ds(btile, BPT)], idxs.at[:, 0, :])
    pltpu.sync_copy(cvt_hbm.at[:, pl.ds(btile, BPT)], cvs.at[:, 0, :])
    pltpu.sync_copy(cidx_hbm, cidx_v.at[pl.ds(0, NCONT)])
    lane = lax.iota(jnp.int32, 16)
    cidx_v[pl.ds(0, 16)] = jnp.where(lane < NCONT, cidx_v[pl.ds(0, 16)], 0)

    row = row2.at[0]

    def stream_row(d):
        off = pl.multiple_of(s * CHK, 128)

        @pl.when(s < NSTREAM)
        def _():
            pltpu.async_copy(
                tabt_hbm.at[pl.ds(d, 1), pl.ds(off, CHK)],
                row2.at[:, pl.ds(off, CHK)],
                sem_s,
            )

        @pl.when(s == NSTREAM)
        def _():
            pltpu.async_copy(
                tail_hbm.at[pl.ds(d, 1), :],
                row2.at[:, pl.ds(NFEAT - 64, 128)],
                sem_s,
            )

    def wait_row(d):
        off = pl.multiple_of(s * CHK, 128)

        @pl.when(s < NSTREAM)
        def _():
            pltpu.make_async_copy(
                tabt_hbm.at[pl.ds(d, 1), pl.ds(off, CHK)],
                row2.at[:, pl.ds(off, CHK)],
                sem_s,
            ).wait()

        @pl.when(s == NSTREAM)
        def _():
            pltpu.make_async_copy(
                tail_hbm.at[pl.ds(d, 1), :],
                row2.at[:, pl.ds(NFEAT - 64, 128)],
                sem_s,
            ).wait()

    stream_row(0)

    def process(d, sbuf, sem_o):
        @pl.when(d >= 2)
        def _():
            pltpu.make_async_copy(
                sbuf,
                out_hbm.at[:, pl.ds(0, 1), pl.ds(btile, BPT)],
                sem_o,
            ).wait()

        cpc = pltpu.async_copy(row.at[cidx_v], cont_sc, sem_c)

        def fire(jj, c2):
            pltpu.async_copy(
                row.at[idxs.at[jj, 0]], sbuf.at[NCONT + jj, 0], sem_g
            )
            return c2

        lax.fori_loop(0, NCATE, fire, 0)

        cpc.wait()
        cvec = cont_sc[pl.ds(0, 16)]
        scal = [cvec[j] for j in range(NCONT)]

        def mulk(k, c2):
            o = k * 16
            for j in range(NCONT):
                sbuf[j, 0, pl.ds(o, 16)] = cvs[j, 0, pl.ds(o, 16)] * scal[j]
            return c2

        lax.fori_loop(0, BPT // 16, mulk, 0)

        def drain(jj, c2):
            pltpu.make_async_copy(
                row.at[idxs.at[0, 0]], sbuf.at[NCONT, 0], sem_g
            ).wait()
            return c2

        lax.fori_loop(0, NCATE, drain, 0)

        plsc.subcore_barrier()

        @pl.when(d + 1 < D)
        def _():
            stream_row(jnp.minimum(d + 1, D - 1))

        pltpu.async_copy(
            sbuf, out_hbm.at[:, pl.ds(d, 1), pl.ds(btile, BPT)], sem_o
        )

    def per_pair(p, carry):
        d0 = 2 * p
        d1 = 2 * p + 1

        wait_row(d0)
        plsc.subcore_barrier()
        process(d0, stage.at[0], sem_o0)

        wait_row(d1)
        plsc.subcore_barrier()
        process(d1, stage.at[1], sem_o1)
        return carry

    lax.fori_loop(0, D // 2, per_pair, 0)

    pltpu.make_async_copy(
        stage.at[0], out_hbm.at[:, pl.ds(0, 1), pl.ds(btile, BPT)], sem_o0
    ).wait()
    pltpu.make_async_copy(
        stage.at[1], out_hbm.at[:, pl.ds(0, 1), pl.ds(btile, BPT)], sem_o1
    ).wait()


def kernel(continuous_value, universal_category_index, emb_table, cont_idx):
    tail = jnp.pad(emb_table.T[:, NFEAT - 64:], ((0, 0), (0, 64)))
    out = _emb_lookup(
        continuous_value.T,
        universal_category_index.astype(jnp.int32).T,
        emb_table.T,
        tail,
        cont_idx.astype(jnp.int32),
    )
    return out.transpose(2, 0, 1)
